# baseline (speedup 1.0000x reference)
import functools

import jax
import jax.numpy as jnp
from jax import lax
from jax.experimental import pallas as pl
from jax.experimental.pallas import tpu as pltpu
from jax.experimental.pallas import tpu_sc as plsc

_NUM_FIELDS = 26
_BATCH = 16384
_EMB = 128

_NC, _NS = 2, 16
_NW = _NC * _NS
_BPW = _BATCH // _NW
_CHUNK = 64
_NCHUNK = _BPW // _CHUNK
_NBUF = 12
_LAG = 4
_NSTEP = _NUM_FIELDS * _NCHUNK

_mesh = plsc.VectorSubcoreMesh(core_axis_name="c", subcore_axis_name="s")


@functools.partial(
    pl.kernel,
    mesh=_mesh,
    out_type=jax.ShapeDtypeStruct((_BATCH, _NUM_FIELDS * _EMB), jnp.float32),
    scratch_types=[
        pltpu.VMEM((_NUM_FIELDS, _NCHUNK, _CHUNK), jnp.int32),
        [pltpu.VMEM((_CHUNK, _EMB), jnp.float32) for _ in range(_NBUF)],
        pltpu.SemaphoreType.DMA,
        [pltpu.SemaphoreType.DMA for _ in range(_NBUF)],
        [pltpu.SemaphoreType.DMA for _ in range(_NBUF)],
    ],
)
def _sc_gather(*refs):
    tables = refs[:_NUM_FIELDS]
    fields = refs[_NUM_FIELDS:2 * _NUM_FIELDS]
    out, idx_v, bufs, isem, gsems, ssems = refs[2 * _NUM_FIELDS:]
    wid = lax.axis_index("s") * _NC + lax.axis_index("c")
    rbase = wid * _NCHUNK
    base = wid * _BPW

    for i in range(_NUM_FIELDS):
        pltpu.async_copy(
            fields[i].at[pl.ds(rbase, _NCHUNK), :], idx_v.at[i], isem)
    _idx_ready = set()

    def wait_idx(i):
        if i not in _idx_ready:
            _idx_ready.add(i)
            pltpu.make_async_copy(
                fields[i].at[pl.ds(rbase, _NCHUNK), :], idx_v.at[i], isem
            ).wait()

    def start_gather(t, b):
        i, j = divmod(t, _NCHUNK)
        wait_idx(i)
        pltpu.async_copy(
            tables[i].at[idx_v.at[i, j]],
            bufs[b], gsems[b],
        )

    def start_scatter(t, b):
        i, j = divmod(t, _NCHUNK)
        pltpu.async_copy(
            bufs[b],
            out.at[pl.ds(base + j * _CHUNK, _CHUNK),
                   pl.ds(i * _EMB, _EMB)],
            ssems[b],
        )

    def wait_gather(b):
        pltpu.make_async_copy(
            tables[0].at[idx_v.at[0, 0]], bufs[b], gsems[b]
        ).wait()

    def wait_scatter(b):
        pltpu.make_async_copy(
            bufs[b], out.at[pl.ds(base, _CHUNK), pl.ds(0, _EMB)], ssems[b]
        ).wait()

    for b in range(_NBUF):
        start_gather(b, b)
    for t in range(_NSTEP):
        b = t % _NBUF
        wait_gather(b)
        start_scatter(t, b)
        if _LAG <= t and t - _LAG + _NBUF < _NSTEP:
            pb = (t - _LAG) % _NBUF
            wait_scatter(pb)
            start_gather(t - _LAG + _NBUF, pb)
    for b in range(_NBUF):
        wait_scatter(b)


def kernel(f0, f1, f2, f3, f4, f5, f6, f7, f8, f9, f10, f11, f12, f13,
           f14, f15, f16, f17, f18, f19, f20, f21, f22, f23, f24, f25,
           T0, T1, T2, T3, T4, T5, T6, T7, T8, T9, T10, T11, T12, T13,
           T14, T15, T16, T17, T18, T19, T20, T21, T22, T23, T24, T25):
    fields = [f0, f1, f2, f3, f4, f5, f6, f7, f8, f9, f10, f11, f12, f13,
              f14, f15, f16, f17, f18, f19, f20, f21, f22, f23, f24, f25]
    tables = [T0, T1, T2, T3, T4, T5, T6, T7, T8, T9, T10, T11, T12, T13,
              T14, T15, T16, T17, T18, T19, T20, T21, T22, T23, T24, T25]
    return _sc_gather(
        *tables,
        *[f.astype(jnp.int32).reshape(_BATCH // _CHUNK, _CHUNK)
          for f in fields])


# speedup vs baseline: 1.1977x; 1.1977x over previous
import functools

import jax
import jax.numpy as jnp
from jax import lax
from jax.experimental import pallas as pl
from jax.experimental.pallas import tpu as pltpu
from jax.experimental.pallas import tpu_sc as plsc

_NUM_FIELDS = 26
_BATCH = 16384
_EMB = 128

_NC, _NS = 2, 16
_NW = _NC * _NS
_BPW = _BATCH // _NW
_CHUNK = 128
_NCHUNK = _BPW // _CHUNK
_NBUF = 7
_LAG = 3
_NSTEP = _NUM_FIELDS * _NCHUNK

_mesh = plsc.VectorSubcoreMesh(core_axis_name="c", subcore_axis_name="s")


@functools.partial(
    pl.kernel,
    mesh=_mesh,
    out_type=jax.ShapeDtypeStruct((_BATCH, _NUM_FIELDS * _EMB), jnp.float32),
    scratch_types=[
        pltpu.VMEM((_NUM_FIELDS, _NCHUNK, _CHUNK), jnp.int32),
        [pltpu.VMEM((_CHUNK, _EMB), jnp.float32) for _ in range(_NBUF)],
        pltpu.SemaphoreType.DMA,
        [pltpu.SemaphoreType.DMA for _ in range(_NBUF)],
        [pltpu.SemaphoreType.DMA for _ in range(_NBUF)],
    ],
)
def _sc_gather(*refs):
    tables = refs[:_NUM_FIELDS]
    fields = refs[_NUM_FIELDS:2 * _NUM_FIELDS]
    out, idx_v, bufs, isem, gsems, ssems = refs[2 * _NUM_FIELDS:]
    wid = lax.axis_index("s") * _NC + lax.axis_index("c")
    rbase = wid * _NCHUNK
    base = wid * _BPW

    for i in range(_NUM_FIELDS):
        pltpu.async_copy(
            fields[i].at[pl.ds(rbase, _NCHUNK), :], idx_v.at[i], isem)
    _idx_ready = set()

    def wait_idx(i):
        if i not in _idx_ready:
            _idx_ready.add(i)
            pltpu.make_async_copy(
                fields[i].at[pl.ds(rbase, _NCHUNK), :], idx_v.at[i], isem
            ).wait()

    def start_gather(t, b):
        i, j = divmod(t, _NCHUNK)
        wait_idx(i)
        pltpu.async_copy(
            tables[i].at[idx_v.at[i, j]],
            bufs[b], gsems[b],
        )

    def start_scatter(t, b):
        i, j = divmod(t, _NCHUNK)
        pltpu.async_copy(
            bufs[b],
            out.at[pl.ds(base + j * _CHUNK, _CHUNK),
                   pl.ds(i * _EMB, _EMB)],
            ssems[b],
        )

    def wait_gather(b):
        pltpu.make_async_copy(
            tables[0].at[idx_v.at[0, 0]], bufs[b], gsems[b]
        ).wait()

    def wait_scatter(b):
        pltpu.make_async_copy(
            bufs[b], out.at[pl.ds(base, _CHUNK), pl.ds(0, _EMB)], ssems[b]
        ).wait()

    for b in range(_NBUF):
        start_gather(b, b)
    for t in range(_NSTEP):
        b = t % _NBUF
        wait_gather(b)
        start_scatter(t, b)
        if _LAG <= t and t - _LAG + _NBUF < _NSTEP:
            pb = (t - _LAG) % _NBUF
            wait_scatter(pb)
            start_gather(t - _LAG + _NBUF, pb)
    for b in range(_NBUF):
        wait_scatter(b)


def kernel(f0, f1, f2, f3, f4, f5, f6, f7, f8, f9, f10, f11, f12, f13,
           f14, f15, f16, f17, f18, f19, f20, f21, f22, f23, f24, f25,
           T0, T1, T2, T3, T4, T5, T6, T7, T8, T9, T10, T11, T12, T13,
           T14, T15, T16, T17, T18, T19, T20, T21, T22, T23, T24, T25):
    fields = [f0, f1, f2, f3, f4, f5, f6, f7, f8, f9, f10, f11, f12, f13,
              f14, f15, f16, f17, f18, f19, f20, f21, f22, f23, f24, f25]
    tables = [T0, T1, T2, T3, T4, T5, T6, T7, T8, T9, T10, T11, T12, T13,
              T14, T15, T16, T17, T18, T19, T20, T21, T22, T23, T24, T25]
    return _sc_gather(
        *tables,
        *[f.astype(jnp.int32).reshape(_BATCH // _CHUNK, _CHUNK)
          for f in fields])


# speedup vs baseline: 1.2129x; 1.0126x over previous
import functools

import jax
import jax.numpy as jnp
from jax import lax
from jax.experimental import pallas as pl
from jax.experimental.pallas import tpu as pltpu
from jax.experimental.pallas import tpu_sc as plsc

_NUM_FIELDS = 26
_BATCH = 16384
_EMB = 128

_NC, _NS = 2, 16
_NW = _NC * _NS
_BPW = _BATCH // _NW
_CHUNK = 128
_NCHUNK = _BPW // _CHUNK
_NBUF = 6
_LAG = 2
_NSTEP = _NUM_FIELDS * _NCHUNK

_mesh = plsc.VectorSubcoreMesh(core_axis_name="c", subcore_axis_name="s")


@functools.partial(
    pl.kernel,
    mesh=_mesh,
    out_type=jax.ShapeDtypeStruct((_BATCH, _NUM_FIELDS * _EMB), jnp.float32),
    scratch_types=[
        pltpu.VMEM((_NUM_FIELDS, _NCHUNK, _CHUNK), jnp.int32),
        [pltpu.VMEM((_CHUNK, _EMB), jnp.float32) for _ in range(_NBUF)],
        pltpu.SemaphoreType.DMA,
        [pltpu.SemaphoreType.DMA for _ in range(_NBUF)],
        [pltpu.SemaphoreType.DMA for _ in range(_NBUF)],
    ],
)
def _sc_gather(*refs):
    tables = refs[:_NUM_FIELDS]
    fields = refs[_NUM_FIELDS:2 * _NUM_FIELDS]
    out, idx_v, bufs, isem, gsems, ssems = refs[2 * _NUM_FIELDS:]
    wid = lax.axis_index("s") * _NC + lax.axis_index("c")
    rbase = wid * _NCHUNK
    base = wid * _BPW

    for i in range(_NUM_FIELDS):
        pltpu.async_copy(
            fields[i].at[pl.ds(rbase, _NCHUNK), :], idx_v.at[i], isem)
    _idx_ready = set()

    def wait_idx(i):
        if i not in _idx_ready:
            _idx_ready.add(i)
            pltpu.make_async_copy(
                fields[i].at[pl.ds(rbase, _NCHUNK), :], idx_v.at[i], isem
            ).wait()

    def start_gather(t, b):
        i, j = divmod(t, _NCHUNK)
        wait_idx(i)
        pltpu.async_copy(
            tables[i].at[idx_v.at[i, j]],
            bufs[b], gsems[b],
        )

    def start_scatter(t, b):
        i, j = divmod(t, _NCHUNK)
        pltpu.async_copy(
            bufs[b],
            out.at[pl.ds(base + j * _CHUNK, _CHUNK),
                   pl.ds(i * _EMB, _EMB)],
            ssems[b],
        )

    def wait_gather(b):
        pltpu.make_async_copy(
            tables[0].at[idx_v.at[0, 0]], bufs[b], gsems[b]
        ).wait()

    def wait_scatter(b):
        pltpu.make_async_copy(
            bufs[b], out.at[pl.ds(base, _CHUNK), pl.ds(0, _EMB)], ssems[b]
        ).wait()

    for b in range(_NBUF):
        start_gather(b, b)
    for t in range(_NSTEP):
        b = t % _NBUF
        wait_gather(b)
        start_scatter(t, b)
        if _LAG <= t and t - _LAG + _NBUF < _NSTEP:
            pb = (t - _LAG) % _NBUF
            wait_scatter(pb)
            start_gather(t - _LAG + _NBUF, pb)
    for b in range(_NBUF):
        wait_scatter(b)


def kernel(f0, f1, f2, f3, f4, f5, f6, f7, f8, f9, f10, f11, f12, f13,
           f14, f15, f16, f17, f18, f19, f20, f21, f22, f23, f24, f25,
           T0, T1, T2, T3, T4, T5, T6, T7, T8, T9, T10, T11, T12, T13,
           T14, T15, T16, T17, T18, T19, T20, T21, T22, T23, T24, T25):
    fields = [f0, f1, f2, f3, f4, f5, f6, f7, f8, f9, f10, f11, f12, f13,
              f14, f15, f16, f17, f18, f19, f20, f21, f22, f23, f24, f25]
    tables = [T0, T1, T2, T3, T4, T5, T6, T7, T8, T9, T10, T11, T12, T13,
              T14, T15, T16, T17, T18, T19, T20, T21, T22, T23, T24, T25]
    return _sc_gather(
        *tables,
        *[f.astype(jnp.int32).reshape(_BATCH // _CHUNK, _CHUNK)
          for f in fields])
